# baseline (speedup 1.0000x reference)
import functools

import jax
import jax.numpy as jnp
from jax import lax
from jax.experimental import pallas as pl
from jax.experimental.pallas import tpu as pltpu
from jax.experimental.pallas import tpu_sc as plsc

_R = 8
_D = 1024
_NW = 32
_NX = 4
_NO = 4
_NP = 2


def kernel(input, pe_table):
    B, S, D = input.shape
    x1 = input.reshape(B * S * D)
    pe1 = pe_table.reshape(pe_table.shape[0] * D)
    rows_per_w = S // _NW
    n_chunks = rows_per_w // _R
    n_units = n_chunks * B
    chunk = _R * D
    mesh = plsc.VectorSubcoreMesh(core_axis_name="c", subcore_axis_name="s")

    scratch = (
        [pltpu.VMEM((chunk,), jnp.float32) for _ in range(_NX + _NO + _NP)]
        + [pltpu.SemaphoreType.DMA for _ in range(_NX + _NO + _NP)]
    )

    @functools.partial(
        pl.kernel,
        mesh=mesh,
        out_type=jax.ShapeDtypeStruct((B * S * D,), jnp.float32),
        scratch_types=scratch,
    )
    def sc_add(x_hbm, pe_hbm, out_hbm, *refs):
        bufs, sems = refs[:_NX + _NO + _NP], refs[_NX + _NO + _NP:]
        xb, ob, pb = bufs[:_NX], bufs[_NX:_NX + _NO], bufs[_NX + _NO:]
        sx, so, sp = sems[:_NX], sems[_NX:_NX + _NO], sems[_NX + _NO:]
        cid = lax.axis_index("c")
        sid = lax.axis_index("s")
        wid = sid * 2 + cid
        s_base = wid * rows_per_w

        def x_off(n):
            c, b = divmod(n, B)
            return (b * S + s_base + c * _R) * D

        def pe_off(c):
            return (s_base + c * _R) * D

        def start_in(n):
            return pltpu.async_copy(
                x_hbm.at[pl.ds(x_off(n), chunk)], xb[n % _NX], sx[n % _NX])

        def start_pe(c):
            return pltpu.async_copy(
                pe_hbm.at[pl.ds(pe_off(c), chunk)], pb[c % _NP], sp[c % _NP])

        pend_x = {n: start_in(n) for n in range(min(_NX - 1, n_units))}
        pend_pe = {0: start_pe(0)}
        pend_out = {}

        for n in range(n_units):
            p = n % _NX
            c = n // B
            o = n % _NO
            if n + _NX - 1 < n_units:
                pend_x[n + _NX - 1] = start_in(n + _NX - 1)
            if n % B == 0 and c + 1 < n_chunks:
                pend_pe[c + 1] = start_pe(c + 1)
            pend_x.pop(n).wait()
            if n % B == 0:
                pend_pe.pop(c).wait()
            if n - _NO in pend_out:
                pend_out.pop(n - _NO).wait()

            xr, pr, orr = xb[p], pb[c % _NP], ob[o]
            pltpu.touch(orr)

            @plsc.parallel_loop(0, chunk, step=16, unroll=8)
            def _(j, xr=xr, pr=pr, orr=orr):
                orr[pl.ds(j, 16)] = xr[pl.ds(j, 16)] + pr[pl.ds(j, 16)]

            pltpu.touch(orr)
            pltpu.touch(xr)
            pltpu.touch(pr)
            pend_out[n] = pltpu.async_copy(
                orr, out_hbm.at[pl.ds(x_off(n), chunk)], so[o])

        for n in sorted(pend_out):
            pend_out.pop(n).wait()

    out = sc_add(x1, pe1)
    return out.reshape(B, S, D)


# speedup vs baseline: 1.3987x; 1.3987x over previous
import functools

import jax
import jax.numpy as jnp
from jax import lax
from jax.experimental import pallas as pl
from jax.experimental.pallas import tpu as pltpu
from jax.experimental.pallas import tpu_sc as plsc

_BS = 512
_S1 = 6656
_R = 8
_NW = 32
_NX = 4
_NO = 4
_NP = 2


def _tc_add_kernel(x_ref, pe_ref, o_ref):
    o_ref[...] = x_ref[...] + pe_ref[...][None, :, :]


def _sc_add(input, pe_table, s1):
    B, S, D = input.shape
    rows_sc = S - s1
    x1 = input.reshape(B * S * D)
    pe1 = pe_table.reshape(pe_table.shape[0] * D)
    rows_per_w = rows_sc // _NW
    n_chunks = rows_per_w // _R
    n_units = n_chunks * B
    chunk = _R * D
    mesh = plsc.VectorSubcoreMesh(core_axis_name="c", subcore_axis_name="s")

    scratch = (
        [pltpu.VMEM((chunk,), jnp.float32) for _ in range(_NX + _NO + _NP)]
        + [pltpu.SemaphoreType.DMA for _ in range(_NX + _NO + _NP)]
    )

    @functools.partial(
        pl.kernel,
        mesh=mesh,
        out_type=jax.ShapeDtypeStruct((B * rows_sc * D,), jnp.float32),
        scratch_types=scratch,
    )
    def sc_add(x_hbm, pe_hbm, out_hbm, *refs):
        bufs, sems = refs[:_NX + _NO + _NP], refs[_NX + _NO + _NP:]
        xb, ob, pb = bufs[:_NX], bufs[_NX:_NX + _NO], bufs[_NX + _NO:]
        sx, so, sp = sems[:_NX], sems[_NX:_NX + _NO], sems[_NX + _NO:]
        cid = lax.axis_index("c")
        sid = lax.axis_index("s")
        wid = sid * 2 + cid
        s_loc = wid * rows_per_w

        def x_off(n):
            c, b = divmod(n, B)
            return (b * S + s1 + s_loc + c * _R) * D

        def o_off(n):
            c, b = divmod(n, B)
            return (b * rows_sc + s_loc + c * _R) * D

        def pe_off(c):
            return (s1 + s_loc + c * _R) * D

        def start_in(n):
            return pltpu.async_copy(
                x_hbm.at[pl.ds(x_off(n), chunk)], xb[n % _NX], sx[n % _NX])

        def start_pe(c):
            return pltpu.async_copy(
                pe_hbm.at[pl.ds(pe_off(c), chunk)], pb[c % _NP], sp[c % _NP])

        pend_x = {n: start_in(n) for n in range(min(_NX - 1, n_units))}
        pend_pe = {0: start_pe(0)}
        pend_out = {}

        for n in range(n_units):
            p = n % _NX
            c = n // B
            o = n % _NO
            if n + _NX - 1 < n_units:
                pend_x[n + _NX - 1] = start_in(n + _NX - 1)
            if n % B == 0 and c + 1 < n_chunks:
                pend_pe[c + 1] = start_pe(c + 1)
            pend_x.pop(n).wait()
            if n % B == 0:
                pend_pe.pop(c).wait()
            if n - _NO in pend_out:
                pend_out.pop(n - _NO).wait()

            xr, pr, orr = xb[p], pb[c % _NP], ob[o]
            pltpu.touch(orr)

            @plsc.parallel_loop(0, chunk, step=16, unroll=8)
            def _(j, xr=xr, pr=pr, orr=orr):
                orr[pl.ds(j, 16)] = xr[pl.ds(j, 16)] + pr[pl.ds(j, 16)]

            pltpu.touch(orr)
            pltpu.touch(xr)
            pltpu.touch(pr)
            pend_out[n] = pltpu.async_copy(
                orr, out_hbm.at[pl.ds(o_off(n), chunk)], so[o])

        for n in sorted(pend_out):
            pend_out.pop(n).wait()

    return sc_add(x1, pe1).reshape(B, rows_sc, D)


def kernel(input, pe_table):
    B, S, D = input.shape
    sc_part = _sc_add(input, pe_table, _S1)
    tc_out = pl.pallas_call(
        _tc_add_kernel,
        grid=(_S1 // _BS,),
        in_specs=[
            pl.BlockSpec((B, _BS, D), lambda s: (0, s, 0)),
            pl.BlockSpec((_BS, D), lambda s: (s, 0)),
        ],
        out_specs=pl.BlockSpec((B, _BS, D), lambda s: (0, s, 0)),
        out_shape=jax.ShapeDtypeStruct((B, S, D), input.dtype),
    )(input, pe_table)
    return lax.dynamic_update_slice(tc_out, sc_part, (0, _S1, 0))


# speedup vs baseline: 1.5335x; 1.0964x over previous
import functools

import jax
import jax.numpy as jnp
from jax import lax
from jax.experimental import pallas as pl
from jax.experimental.pallas import tpu as pltpu
from jax.experimental.pallas import tpu_sc as plsc

_BS = 512
_S1 = 6656
_R = 8
_NW = 32
_NX = 4
_NO = 4
_NP = 2


def _tc_add_kernel(x_ref, pe_ref, o_ref):
    o_ref[...] = x_ref[...] + pe_ref[...][None, :, :]


def _sc_add(input, pe_table, s1):
    B, S, D = input.shape
    rows_sc = S - s1
    x1 = input.reshape(B * S * D)
    pe1 = pe_table.reshape(pe_table.shape[0] * D)
    rows_per_w = rows_sc // _NW
    n_chunks = rows_per_w // _R
    n_units = n_chunks * B
    chunk = _R * D
    mesh = plsc.VectorSubcoreMesh(core_axis_name="c", subcore_axis_name="s")

    scratch = (
        [pltpu.VMEM((chunk,), jnp.float32) for _ in range(_NX + _NO + _NP)]
        + [pltpu.SemaphoreType.DMA for _ in range(_NX + _NO + _NP)]
    )

    @functools.partial(
        pl.kernel,
        mesh=mesh,
        out_type=jax.ShapeDtypeStruct((B * rows_sc * D,), jnp.float32),
        scratch_types=scratch,
    )
    def sc_add(x_hbm, pe_hbm, out_hbm, *refs):
        bufs, sems = refs[:_NX + _NO + _NP], refs[_NX + _NO + _NP:]
        xb, ob, pb = bufs[:_NX], bufs[_NX:_NX + _NO], bufs[_NX + _NO:]
        sx, so, sp = sems[:_NX], sems[_NX:_NX + _NO], sems[_NX + _NO:]
        cid = lax.axis_index("c")
        sid = lax.axis_index("s")
        wid = sid * 2 + cid
        s_loc = wid * rows_per_w

        def x_off(n):
            c, b = divmod(n, B)
            return (b * S + s1 + s_loc + c * _R) * D

        def o_off(n):
            c, b = divmod(n, B)
            return (b * rows_sc + s_loc + c * _R) * D

        def pe_off(c):
            return (s1 + s_loc + c * _R) * D

        def start_in(n):
            return pltpu.async_copy(
                x_hbm.at[pl.ds(x_off(n), chunk)], xb[n % _NX], sx[n % _NX])

        def start_pe(c):
            return pltpu.async_copy(
                pe_hbm.at[pl.ds(pe_off(c), chunk)], pb[c % _NP], sp[c % _NP])

        pend_x = {n: start_in(n) for n in range(min(_NX - 1, n_units))}
        pend_pe = {0: start_pe(0)}
        pend_out = {}

        for n in range(n_units):
            p = n % _NX
            c = n // B
            o = n % _NO
            if n + _NX - 1 < n_units:
                pend_x[n + _NX - 1] = start_in(n + _NX - 1)
            if n % B == 0 and c + 1 < n_chunks:
                pend_pe[c + 1] = start_pe(c + 1)
            pend_x.pop(n).wait()
            if n % B == 0:
                pend_pe.pop(c).wait()
            if n - _NO in pend_out:
                pend_out.pop(n - _NO).wait()

            xr, pr, orr = xb[p], pb[c % _NP], ob[o]
            pltpu.touch(orr)

            @plsc.parallel_loop(0, chunk, step=16, unroll=8)
            def _(j, xr=xr, pr=pr, orr=orr):
                orr[pl.ds(j, 16)] = xr[pl.ds(j, 16)] + pr[pl.ds(j, 16)]

            pltpu.touch(orr)
            pltpu.touch(xr)
            pltpu.touch(pr)
            pend_out[n] = pltpu.async_copy(
                orr, out_hbm.at[pl.ds(o_off(n), chunk)], so[o])

        for n in sorted(pend_out):
            pend_out.pop(n).wait()

    return sc_add(x1, pe1)


def _merge_kernel(tc_ref, sc_ref, o_ref):
    o_ref[...] = sc_ref[...].reshape(o_ref.shape)


def kernel(input, pe_table):
    B, S, D = input.shape
    rows_sc = S - _S1
    sc_part = _sc_add(input, pe_table, _S1)
    tc_out = pl.pallas_call(
        _tc_add_kernel,
        grid=(_S1 // _BS,),
        in_specs=[
            pl.BlockSpec((B, _BS, D), lambda s: (0, s, 0)),
            pl.BlockSpec((_BS, D), lambda s: (s, 0)),
        ],
        out_specs=pl.BlockSpec((B, _BS, D), lambda s: (0, s, 0)),
        out_shape=jax.ShapeDtypeStruct((B, S, D), input.dtype),
    )(input, pe_table)
    nsb = rows_sc // _BS
    return pl.pallas_call(
        _merge_kernel,
        grid=(nsb, B),
        in_specs=[
            pl.BlockSpec(memory_space=pl.ANY),
            pl.BlockSpec((_BS * D,), lambda s, b: (b * nsb + s,)),
        ],
        out_specs=pl.BlockSpec((1, _BS, D), lambda s, b: (b, _S1 // _BS + s, 0)),
        out_shape=jax.ShapeDtypeStruct((B, S, D), input.dtype),
        input_output_aliases={0: 0},
    )(tc_out, sc_part)


# speedup vs baseline: 3.0772x; 2.0066x over previous
import functools

import jax
import jax.numpy as jnp
from jax import lax
from jax.experimental import pallas as pl
from jax.experimental.pallas import tpu as pltpu
from jax.experimental.pallas import tpu_sc as plsc

_BS = 512
_S1 = 6656
_R = 8
_NW = 32
_NX = 4
_NO = 4
_NP = 2


def _tc_add_kernel(x_ref, pe_ref, o_ref):
    o_ref[...] = x_ref[...] + pe_ref[...][None, :, :]


def _merge_kernel(tc_ref, sc_ref, o_ref):
    o_ref[...] = sc_ref[...]


def _sc_add(input, pe_table, s1):
    B, S, D = input.shape
    rows_sc = S - s1
    rows_per_w = rows_sc // _NW
    n_chunks = rows_per_w // _R
    n_units = n_chunks * B
    chunk = _R * D
    mesh = plsc.VectorSubcoreMesh(core_axis_name="c", subcore_axis_name="s")

    scratch = (
        [pltpu.VMEM((_R, D), jnp.float32) for _ in range(_NX + _NO + _NP)]
        + [pltpu.SemaphoreType.DMA for _ in range(_NX + _NO + _NP)]
    )

    @functools.partial(
        pl.kernel,
        mesh=mesh,
        out_type=jax.ShapeDtypeStruct((B, rows_sc, D), jnp.float32),
        scratch_types=scratch,
    )
    def sc_add(x_hbm, pe_hbm, out_hbm, *refs):
        bufs, sems = refs[:_NX + _NO + _NP], refs[_NX + _NO + _NP:]
        xb, ob, pb = bufs[:_NX], bufs[_NX:_NX + _NO], bufs[_NX + _NO:]
        sx, so, sp = sems[:_NX], sems[_NX:_NX + _NO], sems[_NX + _NO:]
        cid = lax.axis_index("c")
        sid = lax.axis_index("s")
        wid = sid * 2 + cid
        s_loc = wid * rows_per_w

        def start_in(n):
            c, b = divmod(n, B)
            return pltpu.async_copy(
                x_hbm.at[b, pl.ds(s1 + s_loc + c * _R, _R), :],
                xb[n % _NX], sx[n % _NX])

        def start_pe(c):
            return pltpu.async_copy(
                pe_hbm.at[pl.ds(s1 + s_loc + c * _R, _R), :],
                pb[c % _NP], sp[c % _NP])

        pend_x = {n: start_in(n) for n in range(min(_NX - 1, n_units))}
        pend_pe = {0: start_pe(0)}
        pend_out = {}

        for n in range(n_units):
            p = n % _NX
            c, b = divmod(n, B)
            o = n % _NO
            if n + _NX - 1 < n_units:
                pend_x[n + _NX - 1] = start_in(n + _NX - 1)
            if n % B == 0 and c + 1 < n_chunks:
                pend_pe[c + 1] = start_pe(c + 1)
            pend_x.pop(n).wait()
            if n % B == 0:
                pend_pe.pop(c).wait()
            if n - _NO in pend_out:
                pend_out.pop(n - _NO).wait()

            xr, pr, orr = xb[p], pb[c % _NP], ob[o]
            pltpu.touch(orr)

            @plsc.parallel_loop(0, chunk, step=16, unroll=8)
            def _(j, xr=xr, pr=pr, orr=orr):
                r = j >> 10
                k = pl.multiple_of(j & (D - 1), 16)
                orr[r, pl.ds(k, 16)] = xr[r, pl.ds(k, 16)] + pr[r, pl.ds(k, 16)]

            pltpu.touch(orr)
            pltpu.touch(xr)
            pltpu.touch(pr)
            pend_out[n] = pltpu.async_copy(
                orr, out_hbm.at[b, pl.ds(s_loc + c * _R, _R), :], so[o])

        for n in sorted(pend_out):
            pend_out.pop(n).wait()

    return sc_add(input, pe_table)


def kernel(input, pe_table):
    B, S, D = input.shape
    rows_sc = S - _S1
    sc_part = _sc_add(input, pe_table, _S1)
    tc_out = pl.pallas_call(
        _tc_add_kernel,
        grid=(_S1 // _BS,),
        in_specs=[
            pl.BlockSpec((B, _BS, D), lambda s: (0, s, 0)),
            pl.BlockSpec((_BS, D), lambda s: (s, 0)),
        ],
        out_specs=pl.BlockSpec((B, _BS, D), lambda s: (0, s, 0)),
        out_shape=jax.ShapeDtypeStruct((B, S, D), input.dtype),
    )(input, pe_table)
    return pl.pallas_call(
        _merge_kernel,
        grid=(rows_sc // _BS, B),
        in_specs=[
            pl.BlockSpec(memory_space=pl.ANY),
            pl.BlockSpec((1, _BS, D), lambda s, b: (b, s, 0)),
        ],
        out_specs=pl.BlockSpec((1, _BS, D), lambda s, b: (b, _S1 // _BS + s, 0)),
        out_shape=jax.ShapeDtypeStruct((B, S, D), input.dtype),
        input_output_aliases={0: 0},
    )(tc_out, sc_part)


# speedup vs baseline: 3.0799x; 1.0009x over previous
import functools

import jax
import jax.numpy as jnp
from jax import lax
from jax.experimental import pallas as pl
from jax.experimental.pallas import tpu as pltpu
from jax.experimental.pallas import tpu_sc as plsc

_BS = 512
_S1 = 6656
_R = 8
_NW = 32
_NX = 4
_NO = 4
_NP = 2


def _tc_add_kernel(x_ref, pe_ref, o_ref):
    o_ref[...] = x_ref[...] + pe_ref[...][None, :, :]


def _merge_kernel(tc_ref, sc_ref, o_ref):
    o_ref[...] = sc_ref[...]


def _sc_add(input, pe_table, s1):
    B, S, D = input.shape
    rows_sc = S - s1
    rows_per_w = rows_sc // _NW
    n_chunks = rows_per_w // _R
    n_units = n_chunks * B
    chunk = _R * D
    mesh = plsc.VectorSubcoreMesh(core_axis_name="c", subcore_axis_name="s")

    scratch = (
        [pltpu.VMEM((_R, D), jnp.float32) for _ in range(_NX + _NO + _NP)]
        + [pltpu.SemaphoreType.DMA for _ in range(_NX + _NO + _NP)]
    )

    @functools.partial(
        pl.kernel,
        mesh=mesh,
        out_type=jax.ShapeDtypeStruct((B, rows_sc, D), jnp.float32),
        scratch_types=scratch,
    )
    def sc_add(x_hbm, pe_hbm, out_hbm, *refs):
        bufs, sems = refs[:_NX + _NO + _NP], refs[_NX + _NO + _NP:]
        xb, ob, pb = bufs[:_NX], bufs[_NX:_NX + _NO], bufs[_NX + _NO:]
        sx, so, sp = sems[:_NX], sems[_NX:_NX + _NO], sems[_NX + _NO:]
        cid = lax.axis_index("c")
        sid = lax.axis_index("s")
        wid = sid * 2 + cid
        s_loc = wid * rows_per_w

        def start_in(n):
            c, b = divmod(n, B)
            return pltpu.async_copy(
                x_hbm.at[b, pl.ds(s1 + s_loc + c * _R, _R), :],
                xb[n % _NX], sx[n % _NX])

        def start_pe(c):
            return pltpu.async_copy(
                pe_hbm.at[pl.ds(s1 + s_loc + c * _R, _R), :],
                pb[c % _NP], sp[c % _NP])

        pend_x = {n: start_in(n) for n in range(min(_NX - 1, n_units))}
        pend_pe = {0: start_pe(0)}
        pend_out = {}

        for n in range(n_units):
            p = n % _NX
            c, b = divmod(n, B)
            o = n % _NO
            if n + _NX - 1 < n_units:
                pend_x[n + _NX - 1] = start_in(n + _NX - 1)
            if n % B == 0 and c + 1 < n_chunks:
                pend_pe[c + 1] = start_pe(c + 1)
            pend_x.pop(n).wait()
            if n % B == 0:
                pend_pe.pop(c).wait()
            if n - _NO in pend_out:
                pend_out.pop(n - _NO).wait()

            xr, pr, orr = xb[p], pb[c % _NP], ob[o]
            pltpu.touch(orr)

            @plsc.parallel_loop(0, chunk, step=16, unroll=8)
            def _(j, xr=xr, pr=pr, orr=orr):
                r = j >> 10
                k = pl.multiple_of(j & (D - 1), 16)
                orr[r, pl.ds(k, 16)] = xr[r, pl.ds(k, 16)] + pr[r, pl.ds(k, 16)]

            pltpu.touch(orr)
            pltpu.touch(xr)
            pltpu.touch(pr)
            pend_out[n] = pltpu.async_copy(
                orr, out_hbm.at[b, pl.ds(s_loc + c * _R, _R), :], so[o])

        for n in sorted(pend_out):
            pend_out.pop(n).wait()

    return sc_add(input, pe_table)


def kernel(input, pe_table):
    B, S, D = input.shape
    rows_sc = S - _S1
    tc_out = pl.pallas_call(
        _tc_add_kernel,
        grid=(_S1 // _BS,),
        in_specs=[
            pl.BlockSpec((B, _BS, D), lambda s: (0, s, 0)),
            pl.BlockSpec((_BS, D), lambda s: (s, 0)),
        ],
        out_specs=pl.BlockSpec((B, _BS, D), lambda s: (0, s, 0)),
        out_shape=jax.ShapeDtypeStruct((B, S, D), input.dtype),
    )(input, pe_table)
    sc_part = _sc_add(input, pe_table, _S1)
    return pl.pallas_call(
        _merge_kernel,
        grid=(rows_sc // _BS, B),
        in_specs=[
            pl.BlockSpec(memory_space=pl.ANY),
            pl.BlockSpec((1, _BS, D), lambda s, b: (b, s, 0)),
        ],
        out_specs=pl.BlockSpec((1, _BS, D), lambda s, b: (b, _S1 // _BS + s, 0)),
        out_shape=jax.ShapeDtypeStruct((B, S, D), input.dtype),
        input_output_aliases={0: 0},
    )(tc_out, sc_part)


# speedup vs baseline: 4.1897x; 1.3603x over previous
import functools

import jax
import jax.numpy as jnp
from jax.experimental import pallas as pl

_BS = 512


def _add_pe_kernel(x_ref, pe_ref, o_ref):
    o_ref[...] = x_ref[...] + pe_ref[...][None, :, :]


@functools.partial(jax.jit, static_argnames=())
def kernel(input, pe_table):
    B, S, D = input.shape
    grid = (S // _BS,)
    return pl.pallas_call(
        _add_pe_kernel,
        grid=grid,
        in_specs=[
            pl.BlockSpec((B, _BS, D), lambda s: (0, s, 0)),
            pl.BlockSpec((_BS, D), lambda s: (s, 0)),
        ],
        out_specs=pl.BlockSpec((B, _BS, D), lambda s: (0, s, 0)),
        out_shape=jax.ShapeDtypeStruct((B, S, D), input.dtype),
    )(input, pe_table)
